# Initial kernel scaffold; baseline (speedup 1.0000x reference)
import jax
import jax.numpy as jnp
from jax.experimental import pallas as pl


def kernel(data, edge_index_d6, edge_type_d6, edge_index_d5, edge_type_d5, child2parent, depth, params):
    raise NotImplementedError("write your pallas kernel here")


# speedup vs baseline: 8.0153x; 8.0153x over previous
import functools

import numpy as np
import jax
import jax.numpy as jnp
from jax import lax
from jax.experimental import pallas as pl
from jax.experimental.pallas import tpu as pltpu
from jax.experimental.pallas import tpu_sc as plsc

_N6 = 10000
_N5 = 2500
_E6 = 320000
_E5 = 80000
_C0 = 128
_C1 = 256
_G = 32
_T = 7

_NW = 32
_K = 128

_NPAD6 = 10240
_NPAD5 = 2560
_NPAD5D = 5120
_EPAD6 = 327680
_EPAD5D = 163840
_EPADP = 16384
_NPADU = 32768

_ZR = 160


def _gn_mats():
    a0 = np.kron(np.eye(_G, dtype=np.float32), np.full((4, 4), 0.25, np.float32))
    a1 = np.kron(np.eye(_G, dtype=np.float32), np.full((8, 8), 0.125, np.float32))
    return a0, a1


_A0_np, _A1_np = _gn_mats()


def _gn_gelu(x, a, g, b):
    mu = jnp.dot(x, a, preferred_element_type=jnp.float32)
    ex2 = jnp.dot(x * x, a, preferred_element_type=jnp.float32)
    var = ex2 - mu * mu
    xn = (x - mu) * lax.rsqrt(var + 1e-5)
    return jax.nn.gelu(xn * g + b)


@functools.lru_cache(maxsize=None)
def _make_prep(nparts, n, c, r, emit_xin):
    wc = (_T + 1) * c

    def body(*refs):
        parts = refs[:nparts]
        a_r, g_r, b_r, w_r, bc_r = refs[nparts:nparts + 5]
        outs = refs[nparts + 5:]
        x = parts[0][...]
        for p in parts[1:]:
            x = x + p[...]
        u = _gn_gelu(x, a_r[...], g_r[...], b_r[...])
        z = jnp.dot(u, w_r[...], preferred_element_type=jnp.float32) + bc_r[...]
        if emit_xin:
            outs[0][...] = x
            outs = outs[1:]
        outs[0][...] = z[:, :c]
        outs[1][...] = z[:, c:]

    in_specs = [pl.BlockSpec((r, c), lambda i: (i, 0)) for _ in range(nparts)]
    in_specs += [
        pl.BlockSpec((c, c), lambda i: (0, 0)),
        pl.BlockSpec((1, c), lambda i: (0, 0)),
        pl.BlockSpec((1, c), lambda i: (0, 0)),
        pl.BlockSpec((c, wc), lambda i: (0, 0)),
        pl.BlockSpec((1, wc), lambda i: (0, 0)),
    ]
    out_shape = []
    out_specs = []
    if emit_xin:
        out_shape.append(jax.ShapeDtypeStruct((n, c), jnp.float32))
        out_specs.append(pl.BlockSpec((r, c), lambda i: (i, 0)))
    out_shape += [jax.ShapeDtypeStruct((n, c), jnp.float32),
                  jax.ShapeDtypeStruct((n, _T * c), jnp.float32)]
    out_specs += [pl.BlockSpec((r, c), lambda i: (i, 0)),
                  pl.BlockSpec((r, _T * c), lambda i: (i, 0))]
    return pl.pallas_call(body, grid=(n // r,), in_specs=in_specs,
                          out_specs=out_specs, out_shape=out_shape)


@functools.lru_cache(maxsize=None)
def _make_sum(nparts, n, c, r):
    def body(*refs):
        x = refs[0][...]
        for p in refs[1:nparts]:
            x = x + p[...]
        refs[nparts][...] = x

    in_specs = [pl.BlockSpec((r, c), lambda i: (i, 0)) for _ in range(nparts)]
    return pl.pallas_call(body, grid=(n // r,), in_specs=in_specs,
                          out_specs=pl.BlockSpec((r, c), lambda i: (i, 0)),
                          out_shape=jax.ShapeDtypeStruct((n, c), jnp.float32))


@functools.lru_cache(maxsize=None)
def _make_down(n, r):
    def body(p0_r, p1_r, ic_r, w_r, bd_r, a_r, g_r, b_r, o_r):
        xm = (p0_r[...] + p1_r[...]) * ic_r[...]
        z = jnp.dot(xm, w_r[...], preferred_element_type=jnp.float32) + bd_r[...]
        o_r[...] = _gn_gelu(z, a_r[...], g_r[...], b_r[...])

    in_specs = [
        pl.BlockSpec((r, _C0), lambda i: (i, 0)),
        pl.BlockSpec((r, _C0), lambda i: (i, 0)),
        pl.BlockSpec((r, _C0), lambda i: (i, 0)),
        pl.BlockSpec((_C0, _C1), lambda i: (0, 0)),
        pl.BlockSpec((1, _C1), lambda i: (0, 0)),
        pl.BlockSpec((_C1, _C1), lambda i: (0, 0)),
        pl.BlockSpec((1, _C1), lambda i: (0, 0)),
        pl.BlockSpec((1, _C1), lambda i: (0, 0)),
    ]
    return pl.pallas_call(body, grid=(n // r,), in_specs=in_specs,
                          out_specs=pl.BlockSpec((r, _C1), lambda i: (i, 0)),
                          out_shape=jax.ShapeDtypeStruct((n, _C1), jnp.float32))


@functools.lru_cache(maxsize=None)
def _make_up(n, r):
    def body(g_r, w_r, bu_r, a_r, gg_r, gb_r, s_r, o_r):
        z = jnp.dot(g_r[...], w_r[...], preferred_element_type=jnp.float32) + bu_r[...]
        o_r[...] = _gn_gelu(z, a_r[...], gg_r[...], gb_r[...]) + s_r[...]

    in_specs = [
        pl.BlockSpec((r, _C1), lambda i: (i, 0)),
        pl.BlockSpec((_C1, _C0), lambda i: (0, 0)),
        pl.BlockSpec((1, _C0), lambda i: (0, 0)),
        pl.BlockSpec((_C0, _C0), lambda i: (0, 0)),
        pl.BlockSpec((1, _C0), lambda i: (0, 0)),
        pl.BlockSpec((1, _C0), lambda i: (0, 0)),
        pl.BlockSpec((r, _C0), lambda i: (i, 0)),
    ]
    return pl.pallas_call(body, grid=(n // r,), in_specs=in_specs,
                          out_specs=pl.BlockSpec((r, _C0), lambda i: (i, 0)),
                          out_shape=jax.ShapeDtypeStruct((n, _C0), jnp.float32))


@functools.lru_cache(maxsize=None)
def _make_agg(npad, epad, c=128):
    rps = npad // 16
    epw = epad // _NW
    nch = epw // _K
    mesh = plsc.VectorSubcoreMesh(core_axis_name="c", subcore_axis_name="s")

    @functools.partial(
        pl.kernel, mesh=mesh,
        out_type=jax.ShapeDtypeStruct((2, npad, c), jnp.float32),
        scratch_types=[
            pltpu.VMEM((_K,), jnp.int32),
            pltpu.VMEM((_K,), jnp.int32),
            pltpu.VMEM((_K, c), jnp.float32),
            pltpu.VMEM((_ZR, c), jnp.float32),
            pltpu.VMEM_SHARED((npad, c), jnp.float32),
            pltpu.SemaphoreType.DMA,
        ],
    )
    def agg(y_hbm, is_hbm, id_hbm, out_hbm, idx_s, idx_d, rows, zbuf, acc, sem):
        cid = lax.axis_index("c")
        sid = lax.axis_index("s")
        w = cid * 16 + sid

        def zfill(i, _):
            r = i // (c // 16)
            col = (i % (c // 16)) * 16
            zbuf[r, pl.ds(col, 16)] = jnp.zeros((16,), jnp.float32)
            return 0
        lax.fori_loop(0, _ZR * (c // 16), zfill, 0)

        def zacc(j, _):
            pltpu.sync_copy(zbuf, acc.at[pl.ds(sid * rps + j * _ZR, _ZR), :])
            return 0
        lax.fori_loop(0, rps // _ZR, zacc, 0)
        plsc.subcore_barrier()

        def body(j, _):
            off = w * epw + j * _K
            pltpu.sync_copy(is_hbm.at[pl.ds(off, _K)], idx_s)
            pltpu.sync_copy(id_hbm.at[pl.ds(off, _K)], idx_d)
            pltpu.async_copy(y_hbm.at[idx_s], rows, sem).wait()
            pltpu.async_copy(rows, acc.at[idx_d], sem, add=True).wait()
            return 0
        lax.fori_loop(0, nch, body, 0)
        plsc.subcore_barrier()

        pltpu.sync_copy(acc.at[pl.ds(sid * rps, rps), :],
                        out_hbm.at[cid, pl.ds(sid * rps, rps), :])

    return agg


@functools.lru_cache(maxsize=None)
def _make_gather(c, npad):
    rpw = npad // _NW
    nch = rpw // _K
    mesh = plsc.VectorSubcoreMesh(core_axis_name="c", subcore_axis_name="s")

    @functools.partial(
        pl.kernel, mesh=mesh,
        out_type=jax.ShapeDtypeStruct((npad, c), jnp.float32),
        scratch_types=[
            pltpu.VMEM((_K,), jnp.int32),
            pltpu.VMEM((_K, c), jnp.float32),
            pltpu.SemaphoreType.DMA,
        ],
    )
    def gat(tab_hbm, idx_hbm, out_hbm, idx_v, rows, sem):
        cid = lax.axis_index("c")
        sid = lax.axis_index("s")
        w = cid * 16 + sid

        def body(j, _):
            off = w * rpw + j * _K
            pltpu.sync_copy(idx_hbm.at[pl.ds(off, _K)], idx_v)
            pltpu.async_copy(tab_hbm.at[idx_v], rows, sem).wait()
            pltpu.sync_copy(rows, out_hbm.at[pl.ds(off, _K), :])
            return 0
        lax.fori_loop(0, nch, body, 0)

    return gat


def _wcat(p):
    cin = p["Ws"].shape[0]
    cout = p["Ws"].shape[1]
    w = jnp.concatenate([p["Ws"], p["W"].transpose(1, 0, 2).reshape(cin, _T * cout)], axis=1)
    b = jnp.concatenate([p["b"], jnp.zeros((_T * cout,), jnp.float32)]).reshape(1, (_T + 1) * cout)
    return w, b


def _pad_i32(x, target, fill):
    return jnp.concatenate([x, jnp.full((target - x.shape[0],), fill, jnp.int32)])


def _resblock(parts, p, n, c, r, npad, epad, s_idx, d_idx, a_np):
    a = jnp.asarray(a_np)
    agg = _make_agg(npad, epad)

    w1, b1 = _wcat(p["c1"])
    g1 = p["n1"]["g"].reshape(1, c)
    gb1 = p["n1"]["b"].reshape(1, c)
    if len(parts) == 1:
        xin = parts[0]
        base1, y1 = _make_prep(1, n, c, r, False)(xin, a, g1, gb1, w1, b1)
    else:
        xin, base1, y1 = _make_prep(len(parts), n, c, r, True)(*parts, a, g1, gb1, w1, b1)
    q = agg(y1.reshape(-1, 128), s_idx, d_idx)

    w2, b2 = _wcat(p["c2"])
    g2 = p["n2"]["g"].reshape(1, c)
    gb2 = p["n2"]["b"].reshape(1, c)
    base2, y2 = _make_prep(3, n, c, r, False)(
        q[0].reshape(-1, c), q[1].reshape(-1, c), base1, a, g2, gb2, w2, b2)
    q2 = agg(y2.reshape(-1, 128), s_idx, d_idx)
    return [xin, q2[0].reshape(-1, c), q2[1].reshape(-1, c), base2]


def kernel(data, edge_index_d6, edge_type_d6, edge_index_d5, edge_type_d5,
           child2parent, depth, params):
    del depth
    s6 = _pad_i32(edge_index_d6[0] * _T + edge_type_d6, _EPAD6, 0)
    d6 = _pad_i32(edge_index_d6[1], _EPAD6, _N6)
    s5a = edge_index_d5[0] * (2 * _T) + 2 * edge_type_d5
    s5 = _pad_i32(jnp.stack([s5a, s5a + 1], -1).reshape(-1), _EPAD5D, 0)
    d5a = 2 * edge_index_d5[1]
    d5 = _pad_i32(jnp.stack([d5a, d5a + 1], -1).reshape(-1), _EPAD5D, 2 * _N5)
    sp = _pad_i32(jnp.arange(_N6, dtype=jnp.int32), _EPADP, 0)
    dp = _pad_i32(child2parent, _EPADP, _N5)
    upa = 2 * child2parent
    up_idx = _pad_i32(jnp.stack([upa, upa + 1], -1).reshape(-1), _NPADU, 0)
    cnt = jax.ops.segment_sum(jnp.ones((_N6,), jnp.float32), child2parent,
                              num_segments=_NPAD5)
    invc = jnp.broadcast_to((1.0 / jnp.clip(cnt, 1.0))[:, None], (_NPAD5, _C0))

    n5e = _NPAD5
    r6, r5 = 1000, 640

    parts = [data]
    for rb in params["enc0"]:
        parts = _resblock(parts, rb, _N6, _C0, r6, _NPAD6, _EPAD6, s6, d6, _A0_np)
    x6 = _make_sum(4, _N6, _C0, r6)(*parts)

    qp = _make_agg(_NPAD5, _EPADP)(x6, sp, dp)
    a1 = jnp.asarray(_A1_np)
    x5 = _make_down(n5e, r5)(
        qp[0], qp[1], invc, params["down"]["W"],
        params["down"]["b"].reshape(1, _C1), a1,
        params["down"]["n"]["g"].reshape(1, _C1),
        params["down"]["n"]["b"].reshape(1, _C1))

    parts = [x5]
    for rb in params["enc1"]:
        parts = _resblock(parts, rb, n5e, _C1, r5, _NPAD5D, _EPAD5D, s5, d5, _A1_np)
    for rb in params["dec0"]:
        parts = _resblock(parts, rb, n5e, _C1, r5, _NPAD5D, _EPAD5D, s5, d5, _A1_np)
    out5 = _make_sum(4, n5e, _C1, r5)(*parts)

    g = _make_gather(128, _NPADU)(out5.reshape(-1, 128), up_idx).reshape(-1, _C1)
    a0 = jnp.asarray(_A0_np)
    xu = _make_up(_N6, r6)(
        g, params["up"]["W"], params["up"]["b"].reshape(1, _C0), a0,
        params["up"]["n"]["g"].reshape(1, _C0),
        params["up"]["n"]["b"].reshape(1, _C0), x6)

    parts = [xu]
    for rb in params["dec1"]:
        parts = _resblock(parts, rb, _N6, _C0, r6, _NPAD6, _EPAD6, s6, d6, _A0_np)
    return _make_sum(4, _N6, _C0, r6)(*parts)


# speedup vs baseline: 23.8528x; 2.9759x over previous
import functools

import numpy as np
import jax
import jax.numpy as jnp
from jax import lax
from jax.experimental import pallas as pl
from jax.experimental.pallas import tpu as pltpu
from jax.experimental.pallas import tpu_sc as plsc

_N6 = 10000
_N5 = 2500
_E6 = 320000
_E5 = 80000
_C0 = 128
_C1 = 256
_G = 32
_T = 7

_NW = 32
_K = 128

_NPAD6 = 10240
_NPAD5 = 2560
_NPAD5D = 5120
_EPAD6 = 327680
_EPAD5D = 163840
_EPADP = 32768
_NPADU = 32768

_ZR = 32


def _gn_mats():
    a0 = np.kron(np.eye(_G, dtype=np.float32), np.full((4, 4), 0.25, np.float32))
    a1 = np.kron(np.eye(_G, dtype=np.float32), np.full((8, 8), 0.125, np.float32))
    return a0, a1


_A0_np, _A1_np = _gn_mats()


def _gn_gelu(x, a, g, b):
    mu = jnp.dot(x, a, preferred_element_type=jnp.float32)
    ex2 = jnp.dot(x * x, a, preferred_element_type=jnp.float32)
    var = ex2 - mu * mu
    xn = (x - mu) * lax.rsqrt(var + 1e-5)
    return jax.nn.gelu(xn * g + b)


@functools.lru_cache(maxsize=None)
def _make_prep(nparts, n, c, r, emit_xin):
    wc = (_T + 1) * c

    def body(*refs):
        parts = refs[:nparts]
        a_r, g_r, b_r, w_r, bc_r = refs[nparts:nparts + 5]
        outs = refs[nparts + 5:]
        x = parts[0][...]
        for p in parts[1:]:
            x = x + p[...]
        u = _gn_gelu(x, a_r[...], g_r[...], b_r[...])
        z = jnp.dot(u, w_r[...], preferred_element_type=jnp.float32) + bc_r[...]
        if emit_xin:
            outs[0][...] = x
            outs = outs[1:]
        outs[0][...] = z[:, :c]
        outs[1][...] = z[:, c:]

    in_specs = [pl.BlockSpec((r, c), lambda i: (i, 0)) for _ in range(nparts)]
    in_specs += [
        pl.BlockSpec((c, c), lambda i: (0, 0)),
        pl.BlockSpec((1, c), lambda i: (0, 0)),
        pl.BlockSpec((1, c), lambda i: (0, 0)),
        pl.BlockSpec((c, wc), lambda i: (0, 0)),
        pl.BlockSpec((1, wc), lambda i: (0, 0)),
    ]
    out_shape = []
    out_specs = []
    if emit_xin:
        out_shape.append(jax.ShapeDtypeStruct((n, c), jnp.float32))
        out_specs.append(pl.BlockSpec((r, c), lambda i: (i, 0)))
    out_shape += [jax.ShapeDtypeStruct((n, c), jnp.float32),
                  jax.ShapeDtypeStruct((n, _T * c), jnp.float32)]
    out_specs += [pl.BlockSpec((r, c), lambda i: (i, 0)),
                  pl.BlockSpec((r, _T * c), lambda i: (i, 0))]
    return pl.pallas_call(body, grid=(n // r,), in_specs=in_specs,
                          out_specs=out_specs, out_shape=out_shape)


@functools.lru_cache(maxsize=None)
def _make_sum(nparts, n, c, r):
    def body(*refs):
        x = refs[0][...]
        for p in refs[1:nparts]:
            x = x + p[...]
        refs[nparts][...] = x

    in_specs = [pl.BlockSpec((r, c), lambda i: (i, 0)) for _ in range(nparts)]
    return pl.pallas_call(body, grid=(n // r,), in_specs=in_specs,
                          out_specs=pl.BlockSpec((r, c), lambda i: (i, 0)),
                          out_shape=jax.ShapeDtypeStruct((n, c), jnp.float32))


@functools.lru_cache(maxsize=None)
def _make_down(n, r):
    def body(p0_r, p1_r, ic_r, w_r, bd_r, a_r, g_r, b_r, o_r):
        xm = (p0_r[...] + p1_r[...]) * ic_r[...]
        z = jnp.dot(xm, w_r[...], preferred_element_type=jnp.float32) + bd_r[...]
        o_r[...] = _gn_gelu(z, a_r[...], g_r[...], b_r[...])

    in_specs = [
        pl.BlockSpec((r, _C0), lambda i: (i, 0)),
        pl.BlockSpec((r, _C0), lambda i: (i, 0)),
        pl.BlockSpec((r, _C0), lambda i: (i, 0)),
        pl.BlockSpec((_C0, _C1), lambda i: (0, 0)),
        pl.BlockSpec((1, _C1), lambda i: (0, 0)),
        pl.BlockSpec((_C1, _C1), lambda i: (0, 0)),
        pl.BlockSpec((1, _C1), lambda i: (0, 0)),
        pl.BlockSpec((1, _C1), lambda i: (0, 0)),
    ]
    return pl.pallas_call(body, grid=(n // r,), in_specs=in_specs,
                          out_specs=pl.BlockSpec((r, _C1), lambda i: (i, 0)),
                          out_shape=jax.ShapeDtypeStruct((n, _C1), jnp.float32))


@functools.lru_cache(maxsize=None)
def _make_up(n, r):
    def body(g_r, w_r, bu_r, a_r, gg_r, gb_r, s_r, o_r):
        z = jnp.dot(g_r[...], w_r[...], preferred_element_type=jnp.float32) + bu_r[...]
        o_r[...] = _gn_gelu(z, a_r[...], gg_r[...], gb_r[...]) + s_r[...]

    in_specs = [
        pl.BlockSpec((r, _C1), lambda i: (i, 0)),
        pl.BlockSpec((_C1, _C0), lambda i: (0, 0)),
        pl.BlockSpec((1, _C0), lambda i: (0, 0)),
        pl.BlockSpec((_C0, _C0), lambda i: (0, 0)),
        pl.BlockSpec((1, _C0), lambda i: (0, 0)),
        pl.BlockSpec((1, _C0), lambda i: (0, 0)),
        pl.BlockSpec((r, _C0), lambda i: (i, 0)),
    ]
    return pl.pallas_call(body, grid=(n // r,), in_specs=in_specs,
                          out_specs=pl.BlockSpec((r, _C0), lambda i: (i, 0)),
                          out_shape=jax.ShapeDtypeStruct((n, _C0), jnp.float32))


@functools.lru_cache(maxsize=None)
def _make_agg(npad, epad, nbuf, c=128):
    rps = npad // 16
    epw = epad // _NW
    nch = epw // _K
    nit = nch // 8
    mesh = plsc.VectorSubcoreMesh(core_axis_name="c", subcore_axis_name="s")

    scr = [pltpu.VMEM((8, _K), jnp.int32), pltpu.VMEM((8, _K), jnp.int32)]
    scr += [pltpu.VMEM((_K, c), jnp.float32)] * nbuf
    scr += [pltpu.VMEM((_ZR, c), jnp.float32),
            pltpu.VMEM_SHARED((npad, c), jnp.float32)]
    scr += [pltpu.SemaphoreType.DMA] * nbuf

    @functools.partial(
        pl.kernel, mesh=mesh,
        out_type=jax.ShapeDtypeStruct((2, npad, c), jnp.float32),
        scratch_types=scr,
    )
    def agg(y_hbm, is_hbm, id_hbm, out_hbm, ixs, ixd, *rest):
        rows = rest[:nbuf]
        zbuf = rest[nbuf]
        acc = rest[nbuf + 1]
        sems = rest[nbuf + 2:]
        cid = lax.axis_index("c")
        sid = lax.axis_index("s")
        w = cid * 16 + sid

        def zfill(i, _):
            rr = i // (c // 16)
            col = (i % (c // 16)) * 16
            zbuf[rr, pl.ds(col, 16)] = jnp.zeros((16,), jnp.float32)
            return 0
        lax.fori_loop(0, _ZR * (c // 16), zfill, 0)

        def zacc(j, _):
            pltpu.sync_copy(zbuf, acc.at[pl.ds(sid * rps + j * _ZR, _ZR), :])
            return 0
        lax.fori_loop(0, rps // _ZR, zacc, 0)
        plsc.subcore_barrier()

        def body(i, _):
            ro = w * nch + i * 8
            pltpu.sync_copy(is_hbm.at[pl.ds(ro, 8), :], ixs)
            pltpu.sync_copy(id_hbm.at[pl.ds(ro, 8), :], ixd)
            for g in range(8 // nbuf):
                gets = [pltpu.async_copy(y_hbm.at[ixs.at[g * nbuf + u]],
                                         rows[u], sems[u])
                        for u in range(nbuf)]
                puts = []
                for u in range(nbuf):
                    gets[u].wait()
                    puts.append(pltpu.async_copy(rows[u],
                                                 acc.at[ixd.at[g * nbuf + u]],
                                                 sems[u], add=True))
                for u in range(nbuf):
                    puts[u].wait()
            return 0
        lax.fori_loop(0, nit, body, 0)
        plsc.subcore_barrier()

        pltpu.sync_copy(acc.at[pl.ds(sid * rps, rps), :],
                        out_hbm.at[cid, pl.ds(sid * rps, rps), :])

    return agg


@functools.lru_cache(maxsize=None)
def _make_gather(npad, c=128):
    rpw = npad // _NW
    nch = rpw // _K
    nit = nch // 8
    mesh = plsc.VectorSubcoreMesh(core_axis_name="c", subcore_axis_name="s")

    @functools.partial(
        pl.kernel, mesh=mesh,
        out_type=jax.ShapeDtypeStruct((npad, c), jnp.float32),
        scratch_types=[
            pltpu.VMEM((8, _K), jnp.int32),
            pltpu.VMEM((_K, c), jnp.float32),
            pltpu.VMEM((_K, c), jnp.float32),
            pltpu.VMEM((_K, c), jnp.float32),
            pltpu.VMEM((_K, c), jnp.float32),
            pltpu.SemaphoreType.DMA,
            pltpu.SemaphoreType.DMA,
            pltpu.SemaphoreType.DMA,
            pltpu.SemaphoreType.DMA,
        ],
    )
    def gat(tab_hbm, idx_hbm, out_hbm, ixs, r0, r1, r2, r3, s0, s1, s2, s3):
        cid = lax.axis_index("c")
        sid = lax.axis_index("s")
        w = cid * 16 + sid
        rows = (r0, r1, r2, r3)
        sems = (s0, s1, s2, s3)

        def body(i, _):
            ro = w * nch + i * 8
            pltpu.sync_copy(idx_hbm.at[pl.ds(ro, 8), :], ixs)
            for g in range(2):
                gets = [pltpu.async_copy(tab_hbm.at[ixs.at[g * 4 + u]],
                                         rows[u], sems[u])
                        for u in range(4)]
                puts = []
                for u in range(4):
                    gets[u].wait()
                    puts.append(pltpu.async_copy(
                        rows[u],
                        out_hbm.at[pl.ds((ro + g * 4 + u) * _K, _K), :],
                        sems[u]))
                for u in range(4):
                    puts[u].wait()
            return 0
        lax.fori_loop(0, nit, body, 0)

    return gat


def _wcat(p):
    cin = p["Ws"].shape[0]
    cout = p["Ws"].shape[1]
    w = jnp.concatenate([p["Ws"], p["W"].transpose(1, 0, 2).reshape(cin, _T * cout)], axis=1)
    b = jnp.concatenate([p["b"], jnp.zeros((_T * cout,), jnp.float32)]).reshape(1, (_T + 1) * cout)
    return w, b


def _pad_i32(x, target, fill_lo, fill_n):
    npd = target - x.shape[0]
    pad = fill_lo + jnp.arange(npd, dtype=jnp.int32) % fill_n
    return jnp.concatenate([x, pad]).reshape(-1, _K)


def _resblock(parts, p, n, c, r, npad, epad, nbuf, s_idx, d_idx, a_np):
    a = jnp.asarray(a_np)
    agg = _make_agg(npad, epad, nbuf)

    w1, b1 = _wcat(p["c1"])
    g1 = p["n1"]["g"].reshape(1, c)
    gb1 = p["n1"]["b"].reshape(1, c)
    if len(parts) == 1:
        xin = parts[0]
        base1, y1 = _make_prep(1, n, c, r, False)(xin, a, g1, gb1, w1, b1)
    else:
        xin, base1, y1 = _make_prep(len(parts), n, c, r, True)(*parts, a, g1, gb1, w1, b1)
    q = agg(y1.reshape(-1, 128), s_idx, d_idx)

    w2, b2 = _wcat(p["c2"])
    g2 = p["n2"]["g"].reshape(1, c)
    gb2 = p["n2"]["b"].reshape(1, c)
    base2, y2 = _make_prep(3, n, c, r, False)(
        q[0].reshape(-1, c), q[1].reshape(-1, c), base1, a, g2, gb2, w2, b2)
    q2 = agg(y2.reshape(-1, 128), s_idx, d_idx)
    return [xin, q2[0].reshape(-1, c), q2[1].reshape(-1, c), base2]


def kernel(data, edge_index_d6, edge_type_d6, edge_index_d5, edge_type_d5,
           child2parent, depth, params):
    del depth
    s6 = _pad_i32(edge_index_d6[0] * _T + edge_type_d6, _EPAD6, 0, 512)
    d6 = _pad_i32(edge_index_d6[1], _EPAD6, _N6, _NPAD6 - _N6)
    s5a = edge_index_d5[0] * (2 * _T) + 2 * edge_type_d5
    s5 = _pad_i32(jnp.stack([s5a, s5a + 1], -1).reshape(-1), _EPAD5D, 0, 512)
    d5a = 2 * edge_index_d5[1]
    d5 = _pad_i32(jnp.stack([d5a, d5a + 1], -1).reshape(-1), _EPAD5D,
                  2 * _N5, _NPAD5D - 2 * _N5)
    sp = _pad_i32(jnp.arange(_N6, dtype=jnp.int32), _EPADP, 0, 512)
    dp = _pad_i32(child2parent, _EPADP, _N5, _NPAD5 - _N5)
    upa = 2 * child2parent
    up_idx = _pad_i32(jnp.stack([upa, upa + 1], -1).reshape(-1), _NPADU, 0, 512)
    cnt = jax.ops.segment_sum(jnp.ones((_N6,), jnp.float32), child2parent,
                              num_segments=_NPAD5)
    invc = jnp.broadcast_to((1.0 / jnp.clip(cnt, 1.0))[:, None], (_NPAD5, _C0))

    n5e = _NPAD5
    r6, r5 = 1000, 640

    parts = [data]
    for rb in params["enc0"]:
        parts = _resblock(parts, rb, _N6, _C0, r6, _NPAD6, _EPAD6, 2, s6, d6, _A0_np)
    x6 = _make_sum(4, _N6, _C0, r6)(*parts)

    qp = _make_agg(_NPAD5, _EPADP, 4)(x6, sp, dp)
    a1 = jnp.asarray(_A1_np)
    x5 = _make_down(n5e, r5)(
        qp[0], qp[1], invc, params["down"]["W"],
        params["down"]["b"].reshape(1, _C1), a1,
        params["down"]["n"]["g"].reshape(1, _C1),
        params["down"]["n"]["b"].reshape(1, _C1))

    parts = [x5]
    for rb in params["enc1"]:
        parts = _resblock(parts, rb, n5e, _C1, r5, _NPAD5D, _EPAD5D, 4, s5, d5, _A1_np)
    for rb in params["dec0"]:
        parts = _resblock(parts, rb, n5e, _C1, r5, _NPAD5D, _EPAD5D, 4, s5, d5, _A1_np)
    out5 = _make_sum(4, n5e, _C1, r5)(*parts)

    g = _make_gather(_NPADU)(out5.reshape(-1, 128), up_idx).reshape(-1, _C1)
    a0 = jnp.asarray(_A0_np)
    xu = _make_up(_N6, r6)(
        g, params["up"]["W"], params["up"]["b"].reshape(1, _C0), a0,
        params["up"]["n"]["g"].reshape(1, _C0),
        params["up"]["n"]["b"].reshape(1, _C0), x6)

    parts = [xu]
    for rb in params["dec1"]:
        parts = _resblock(parts, rb, _N6, _C0, r6, _NPAD6, _EPAD6, 2, s6, d6, _A0_np)
    return _make_sum(4, _N6, _C0, r6)(*parts)


# speedup vs baseline: 28.1716x; 1.1811x over previous
import functools

import numpy as np
import jax
import jax.numpy as jnp
from jax import lax
from jax.experimental import pallas as pl
from jax.experimental.pallas import tpu as pltpu
from jax.experimental.pallas import tpu_sc as plsc

_N6 = 10000
_N5 = 2500
_E6 = 320000
_E5 = 80000
_C0 = 128
_C1 = 256
_G = 32
_T = 7

_NW = 32
_K = 128

_NPAD6 = 10240
_NPAD5 = 2560
_NPAD5D = 5120
_EPAD6 = 327680
_EPAD5D = 163840
_EPADP = 32768
_NPADU = 32768

_ZR = 32


def _gn_mats():
    a0 = np.kron(np.eye(_G, dtype=np.float32), np.full((4, 4), 0.25, np.float32))
    a1 = np.kron(np.eye(_G, dtype=np.float32), np.full((8, 8), 0.125, np.float32))
    return a0, a1


_A0_np, _A1_np = _gn_mats()


def _gn_gelu(x, a, g, b):
    mu = jnp.dot(x, a, preferred_element_type=jnp.float32)
    ex2 = jnp.dot(x * x, a, preferred_element_type=jnp.float32)
    var = ex2 - mu * mu
    xn = (x - mu) * lax.rsqrt(var + 1e-5)
    return jax.nn.gelu(xn * g + b)


@functools.lru_cache(maxsize=None)
def _make_prep(nparts, n, c, r, emit_xin):
    wc = (_T + 1) * c

    def body(*refs):
        parts = refs[:nparts]
        a_r, g_r, b_r, w_r, bc_r = refs[nparts:nparts + 5]
        outs = refs[nparts + 5:]
        x = parts[0][...]
        for p in parts[1:]:
            x = x + p[...]
        u = _gn_gelu(x, a_r[...], g_r[...], b_r[...])
        z = jnp.dot(u, w_r[...], preferred_element_type=jnp.float32) + bc_r[...]
        if emit_xin:
            outs[0][...] = x
            outs = outs[1:]
        outs[0][...] = z[:, :c]
        outs[1][...] = z[:, c:]

    in_specs = [pl.BlockSpec((r, c), lambda i: (i, 0)) for _ in range(nparts)]
    in_specs += [
        pl.BlockSpec((c, c), lambda i: (0, 0)),
        pl.BlockSpec((1, c), lambda i: (0, 0)),
        pl.BlockSpec((1, c), lambda i: (0, 0)),
        pl.BlockSpec((c, wc), lambda i: (0, 0)),
        pl.BlockSpec((1, wc), lambda i: (0, 0)),
    ]
    out_shape = []
    out_specs = []
    if emit_xin:
        out_shape.append(jax.ShapeDtypeStruct((n, c), jnp.float32))
        out_specs.append(pl.BlockSpec((r, c), lambda i: (i, 0)))
    out_shape += [jax.ShapeDtypeStruct((n, c), jnp.float32),
                  jax.ShapeDtypeStruct((n, _T * c), jnp.float32)]
    out_specs += [pl.BlockSpec((r, c), lambda i: (i, 0)),
                  pl.BlockSpec((r, _T * c), lambda i: (i, 0))]
    return pl.pallas_call(body, grid=(n // r,), in_specs=in_specs,
                          out_specs=out_specs, out_shape=out_shape)


@functools.lru_cache(maxsize=None)
def _make_sum(nparts, n, c, r):
    def body(*refs):
        x = refs[0][...]
        for p in refs[1:nparts]:
            x = x + p[...]
        refs[nparts][...] = x

    in_specs = [pl.BlockSpec((r, c), lambda i: (i, 0)) for _ in range(nparts)]
    return pl.pallas_call(body, grid=(n // r,), in_specs=in_specs,
                          out_specs=pl.BlockSpec((r, c), lambda i: (i, 0)),
                          out_shape=jax.ShapeDtypeStruct((n, c), jnp.float32))


@functools.lru_cache(maxsize=None)
def _make_down(n, r):
    def body(p0_r, p1_r, ic_r, w_r, bd_r, a_r, g_r, b_r, o_r):
        xm = (p0_r[...] + p1_r[...]) * ic_r[...]
        z = jnp.dot(xm, w_r[...], preferred_element_type=jnp.float32) + bd_r[...]
        o_r[...] = _gn_gelu(z, a_r[...], g_r[...], b_r[...])

    in_specs = [
        pl.BlockSpec((r, _C0), lambda i: (i, 0)),
        pl.BlockSpec((r, _C0), lambda i: (i, 0)),
        pl.BlockSpec((r, _C0), lambda i: (i, 0)),
        pl.BlockSpec((_C0, _C1), lambda i: (0, 0)),
        pl.BlockSpec((1, _C1), lambda i: (0, 0)),
        pl.BlockSpec((_C1, _C1), lambda i: (0, 0)),
        pl.BlockSpec((1, _C1), lambda i: (0, 0)),
        pl.BlockSpec((1, _C1), lambda i: (0, 0)),
    ]
    return pl.pallas_call(body, grid=(n // r,), in_specs=in_specs,
                          out_specs=pl.BlockSpec((r, _C1), lambda i: (i, 0)),
                          out_shape=jax.ShapeDtypeStruct((n, _C1), jnp.float32))


@functools.lru_cache(maxsize=None)
def _make_up(n, r):
    def body(g_r, w_r, bu_r, a_r, gg_r, gb_r, s_r, o_r):
        z = jnp.dot(g_r[...], w_r[...], preferred_element_type=jnp.float32) + bu_r[...]
        o_r[...] = _gn_gelu(z, a_r[...], gg_r[...], gb_r[...]) + s_r[...]

    in_specs = [
        pl.BlockSpec((r, _C1), lambda i: (i, 0)),
        pl.BlockSpec((_C1, _C0), lambda i: (0, 0)),
        pl.BlockSpec((1, _C0), lambda i: (0, 0)),
        pl.BlockSpec((_C0, _C0), lambda i: (0, 0)),
        pl.BlockSpec((1, _C0), lambda i: (0, 0)),
        pl.BlockSpec((1, _C0), lambda i: (0, 0)),
        pl.BlockSpec((r, _C0), lambda i: (i, 0)),
    ]
    return pl.pallas_call(body, grid=(n // r,), in_specs=in_specs,
                          out_specs=pl.BlockSpec((r, _C0), lambda i: (i, 0)),
                          out_shape=jax.ShapeDtypeStruct((n, _C0), jnp.float32))


@functools.lru_cache(maxsize=None)
def _make_agg(npad, epad, nbuf, c=128):
    rps = npad // 16
    epw = epad // _NW
    nch = epw // _K
    nit = nch // 8
    mesh = plsc.VectorSubcoreMesh(core_axis_name="c", subcore_axis_name="s")

    scr = [pltpu.VMEM((8, _K), jnp.int32), pltpu.VMEM((8, _K), jnp.int32),
           pltpu.VMEM((8, _K), jnp.int32), pltpu.VMEM((8, _K), jnp.int32)]
    scr += [pltpu.VMEM((_K, c), jnp.float32)] * nbuf
    scr += [pltpu.VMEM_SHARED((npad, c), jnp.float32)]
    scr += [pltpu.SemaphoreType.DMA] * nbuf
    scr += [pltpu.SemaphoreType.DMA]

    @functools.partial(
        pl.kernel, mesh=mesh,
        out_type=jax.ShapeDtypeStruct((2, npad, c), jnp.float32),
        scratch_types=scr,
    )
    def agg(y_hbm, is_hbm, id_hbm, out_hbm, *rest):
        ixs = (rest[0], rest[1])
        ixd = (rest[2], rest[3])
        rows = rest[4:4 + nbuf]
        acc = rest[4 + nbuf]
        sems = rest[5 + nbuf:5 + 2 * nbuf]
        isem = rest[5 + 2 * nbuf]
        cid = lax.axis_index("c")
        sid = lax.axis_index("s")
        w = cid * 16 + sid

        def zfill(i, _):
            rr = i // (c // 16)
            col = (i % (c // 16)) * 16
            for u in range(nbuf):
                rows[u][rr, pl.ds(col, 16)] = jnp.zeros((16,), jnp.float32)
            return 0
        lax.fori_loop(0, 32 * (c // 16), zfill, 0)

        nz = rps // 32

        def zacc(j, _):
            zs = [pltpu.async_copy(
                rows[u].at[pl.ds(0, 32), :],
                acc.at[pl.ds(sid * rps + (j * nbuf + u) * 32, 32), :],
                sems[u]) for u in range(nbuf)]
            for z in zs:
                z.wait()
            return 0
        lax.fori_loop(0, nz // nbuf, zacc, 0)
        for t in range((nz // nbuf) * nbuf, nz):
            pltpu.sync_copy(rows[0].at[pl.ds(0, 32), :],
                            acc.at[pl.ds(sid * rps + t * 32, 32), :])
        plsc.subcore_barrier()

        def load_idx(i, slot):
            ro = w * nch + i * 8
            a = pltpu.async_copy(is_hbm.at[pl.ds(ro, 8), :], ixs[slot], isem)
            b = pltpu.async_copy(id_hbm.at[pl.ds(ro, 8), :], ixd[slot], isem)
            return a, b

        a0, b0 = load_idx(0, 0)
        a0.wait()
        b0.wait()

        def body(i, _):
            @pl.when(i + 1 < nit)
            def _():
                ro = w * nch + (i + 1) * 8
                pltpu.async_copy(is_hbm.at[pl.ds(ro, 8), :], ixs[1], isem)
                pltpu.async_copy(id_hbm.at[pl.ds(ro, 8), :], ixd[1], isem)

            gets = {}
            puts = {}
            for u in range(nbuf):
                gets[u] = pltpu.async_copy(y_hbm.at[ixs[0].at[u]], rows[u],
                                           sems[u])
            for k in range(8):
                u = k % nbuf
                gets[k].wait()
                puts[k] = pltpu.async_copy(rows[u], acc.at[ixd[0].at[k]],
                                           sems[u], add=True)
                if k + nbuf < 8:
                    puts[k].wait()
                    gets[k + nbuf] = pltpu.async_copy(
                        y_hbm.at[ixs[0].at[k + nbuf]], rows[u], sems[u])
            for k in range(8 - nbuf, 8):
                puts[k].wait()

            @pl.when(i + 1 < nit)
            def _():
                pltpu.make_async_copy(is_hbm.at[pl.ds(0, 8), :], ixs[1], isem).wait()
                pltpu.make_async_copy(id_hbm.at[pl.ds(0, 8), :], ixd[1], isem).wait()

                def rot(i2, _):
                    rr = i2 // 8
                    col = (i2 % 8) * 16
                    ixs[0][rr, pl.ds(col, 16)] = ixs[1][rr, pl.ds(col, 16)]
                    ixd[0][rr, pl.ds(col, 16)] = ixd[1][rr, pl.ds(col, 16)]
                    return 0
                lax.fori_loop(0, 64, rot, 0)
            return 0
        lax.fori_loop(0, nit, body, 0)
        plsc.subcore_barrier()

        pltpu.sync_copy(acc.at[pl.ds(sid * rps, rps), :],
                        out_hbm.at[cid, pl.ds(sid * rps, rps), :])

    return agg


@functools.lru_cache(maxsize=None)
def _make_gather(npad, c=128):
    rpw = npad // _NW
    nch = rpw // _K
    nit = nch // 8
    mesh = plsc.VectorSubcoreMesh(core_axis_name="c", subcore_axis_name="s")

    @functools.partial(
        pl.kernel, mesh=mesh,
        out_type=jax.ShapeDtypeStruct((npad, c), jnp.float32),
        scratch_types=[
            pltpu.VMEM((8, _K), jnp.int32),
            pltpu.VMEM((_K, c), jnp.float32),
            pltpu.VMEM((_K, c), jnp.float32),
            pltpu.VMEM((_K, c), jnp.float32),
            pltpu.VMEM((_K, c), jnp.float32),
            pltpu.SemaphoreType.DMA,
            pltpu.SemaphoreType.DMA,
            pltpu.SemaphoreType.DMA,
            pltpu.SemaphoreType.DMA,
        ],
    )
    def gat(tab_hbm, idx_hbm, out_hbm, ixs, r0, r1, r2, r3, s0, s1, s2, s3):
        cid = lax.axis_index("c")
        sid = lax.axis_index("s")
        w = cid * 16 + sid
        rows = (r0, r1, r2, r3)
        sems = (s0, s1, s2, s3)

        def body(i, _):
            ro = w * nch + i * 8
            pltpu.sync_copy(idx_hbm.at[pl.ds(ro, 8), :], ixs)
            for g in range(2):
                gets = [pltpu.async_copy(tab_hbm.at[ixs.at[g * 4 + u]],
                                         rows[u], sems[u])
                        for u in range(4)]
                puts = []
                for u in range(4):
                    gets[u].wait()
                    puts.append(pltpu.async_copy(
                        rows[u],
                        out_hbm.at[pl.ds((ro + g * 4 + u) * _K, _K), :],
                        sems[u]))
                for u in range(4):
                    puts[u].wait()
            return 0
        lax.fori_loop(0, nit, body, 0)

    return gat


def _wcat(p):
    cin = p["Ws"].shape[0]
    cout = p["Ws"].shape[1]
    w = jnp.concatenate([p["Ws"], p["W"].transpose(1, 0, 2).reshape(cin, _T * cout)], axis=1)
    b = jnp.concatenate([p["b"], jnp.zeros((_T * cout,), jnp.float32)]).reshape(1, (_T + 1) * cout)
    return w, b


def _pad_i32(x, target, fill_lo, fill_n):
    npd = target - x.shape[0]
    pad = fill_lo + jnp.arange(npd, dtype=jnp.int32) % fill_n
    return jnp.concatenate([x, pad]).reshape(-1, _K)


def _resblock(parts, p, n, c, r, npad, epad, nbuf, s_idx, d_idx, a_np):
    a = jnp.asarray(a_np)
    agg = _make_agg(npad, epad, nbuf)

    w1, b1 = _wcat(p["c1"])
    g1 = p["n1"]["g"].reshape(1, c)
    gb1 = p["n1"]["b"].reshape(1, c)
    if len(parts) == 1:
        xin = parts[0]
        base1, y1 = _make_prep(1, n, c, r, False)(xin, a, g1, gb1, w1, b1)
    else:
        xin, base1, y1 = _make_prep(len(parts), n, c, r, True)(*parts, a, g1, gb1, w1, b1)
    q = agg(y1.reshape(-1, 128), s_idx, d_idx)

    w2, b2 = _wcat(p["c2"])
    g2 = p["n2"]["g"].reshape(1, c)
    gb2 = p["n2"]["b"].reshape(1, c)
    base2, y2 = _make_prep(3, n, c, r, False)(
        q[0].reshape(-1, c), q[1].reshape(-1, c), base1, a, g2, gb2, w2, b2)
    q2 = agg(y2.reshape(-1, 128), s_idx, d_idx)
    return [xin, q2[0].reshape(-1, c), q2[1].reshape(-1, c), base2]


def kernel(data, edge_index_d6, edge_type_d6, edge_index_d5, edge_type_d5,
           child2parent, depth, params):
    del depth
    s6 = _pad_i32(edge_index_d6[0] * _T + edge_type_d6, _EPAD6, 0, 512)
    d6 = _pad_i32(edge_index_d6[1], _EPAD6, _N6, _NPAD6 - _N6)
    s5a = edge_index_d5[0] * (2 * _T) + 2 * edge_type_d5
    s5 = _pad_i32(jnp.stack([s5a, s5a + 1], -1).reshape(-1), _EPAD5D, 0, 512)
    d5a = 2 * edge_index_d5[1]
    d5 = _pad_i32(jnp.stack([d5a, d5a + 1], -1).reshape(-1), _EPAD5D,
                  2 * _N5, _NPAD5D - 2 * _N5)
    sp = _pad_i32(jnp.arange(_N6, dtype=jnp.int32), _EPADP, 0, 512)
    dp = _pad_i32(child2parent, _EPADP, _N5, _NPAD5 - _N5)
    upa = 2 * child2parent
    up_idx = _pad_i32(jnp.stack([upa, upa + 1], -1).reshape(-1), _NPADU, 0, 512)
    cnt = jax.ops.segment_sum(jnp.ones((_N6,), jnp.float32), child2parent,
                              num_segments=_NPAD5)
    invc = jnp.broadcast_to((1.0 / jnp.clip(cnt, 1.0))[:, None], (_NPAD5, _C0))

    n5e = _NPAD5
    r6, r5 = 1000, 640

    parts = [data]
    for rb in params["enc0"]:
        parts = _resblock(parts, rb, _N6, _C0, r6, _NPAD6, _EPAD6, 2, s6, d6, _A0_np)
    x6 = _make_sum(4, _N6, _C0, r6)(*parts)

    qp = _make_agg(_NPAD5, _EPADP, 4)(x6, sp, dp)
    a1 = jnp.asarray(_A1_np)
    x5 = _make_down(n5e, r5)(
        qp[0], qp[1], invc, params["down"]["W"],
        params["down"]["b"].reshape(1, _C1), a1,
        params["down"]["n"]["g"].reshape(1, _C1),
        params["down"]["n"]["b"].reshape(1, _C1))

    parts = [x5]
    for rb in params["enc1"]:
        parts = _resblock(parts, rb, n5e, _C1, r5, _NPAD5D, _EPAD5D, 4, s5, d5, _A1_np)
    for rb in params["dec0"]:
        parts = _resblock(parts, rb, n5e, _C1, r5, _NPAD5D, _EPAD5D, 4, s5, d5, _A1_np)
    out5 = _make_sum(4, n5e, _C1, r5)(*parts)

    g = _make_gather(_NPADU)(out5.reshape(-1, 128), up_idx).reshape(-1, _C1)
    a0 = jnp.asarray(_A0_np)
    xu = _make_up(_N6, r6)(
        g, params["up"]["W"], params["up"]["b"].reshape(1, _C0), a0,
        params["up"]["n"]["g"].reshape(1, _C0),
        params["up"]["n"]["b"].reshape(1, _C0), x6)

    parts = [xu]
    for rb in params["dec1"]:
        parts = _resblock(parts, rb, _N6, _C0, r6, _NPAD6, _EPAD6, 2, s6, d6, _A0_np)
    return _make_sum(4, _N6, _C0, r6)(*parts)


# speedup vs baseline: 29.1871x; 1.0360x over previous
import functools

import numpy as np
import jax
import jax.numpy as jnp
from jax import lax
from jax.experimental import pallas as pl
from jax.experimental.pallas import tpu as pltpu
from jax.experimental.pallas import tpu_sc as plsc

_N6 = 10000
_N5 = 2500
_E6 = 320000
_E5 = 80000
_C0 = 128
_C1 = 256
_G = 32
_T = 7

_NW = 32
_K = 128

_NPAD6 = 10240
_NPAD5 = 2560
_NPAD5D = 5120
_EPAD6 = 327680
_EPAD5D = 163840
_EPADP = 32768
_NPADU = 32768

_ZR = 32


def _gn_mats():
    a0 = np.kron(np.eye(_G, dtype=np.float32), np.full((4, 4), 0.25, np.float32))
    a1 = np.kron(np.eye(_G, dtype=np.float32), np.full((8, 8), 0.125, np.float32))
    return a0, a1


_A0_np, _A1_np = _gn_mats()


def _gn_gelu(x, a, g, b):
    mu = jnp.dot(x, a, preferred_element_type=jnp.float32)
    ex2 = jnp.dot(x * x, a, preferred_element_type=jnp.float32)
    var = ex2 - mu * mu
    xn = (x - mu) * lax.rsqrt(var + 1e-5)
    return jax.nn.gelu(xn * g + b)


@functools.lru_cache(maxsize=None)
def _make_prep(nparts, n, c, r, emit_xin):
    wc = (_T + 1) * c

    def body(*refs):
        parts = refs[:nparts]
        a_r, g_r, b_r, w_r, bc_r = refs[nparts:nparts + 5]
        outs = refs[nparts + 5:]
        x = parts[0][...]
        for p in parts[1:]:
            x = x + p[...]
        u = _gn_gelu(x, a_r[...], g_r[...], b_r[...])
        z = jnp.dot(u, w_r[...], preferred_element_type=jnp.float32) + bc_r[...]
        if emit_xin:
            outs[0][...] = x
            outs = outs[1:]
        outs[0][...] = z[:, :c]
        outs[1][...] = z[:, c:]

    in_specs = [pl.BlockSpec((r, c), lambda i: (i, 0)) for _ in range(nparts)]
    in_specs += [
        pl.BlockSpec((c, c), lambda i: (0, 0)),
        pl.BlockSpec((1, c), lambda i: (0, 0)),
        pl.BlockSpec((1, c), lambda i: (0, 0)),
        pl.BlockSpec((c, wc), lambda i: (0, 0)),
        pl.BlockSpec((1, wc), lambda i: (0, 0)),
    ]
    out_shape = []
    out_specs = []
    if emit_xin:
        out_shape.append(jax.ShapeDtypeStruct((n, c), jnp.float32))
        out_specs.append(pl.BlockSpec((r, c), lambda i: (i, 0)))
    out_shape += [jax.ShapeDtypeStruct((n, c), jnp.float32),
                  jax.ShapeDtypeStruct((n, _T * c), jnp.float32)]
    out_specs += [pl.BlockSpec((r, c), lambda i: (i, 0)),
                  pl.BlockSpec((r, _T * c), lambda i: (i, 0))]
    return pl.pallas_call(body, grid=(n // r,), in_specs=in_specs,
                          out_specs=out_specs, out_shape=out_shape)


@functools.lru_cache(maxsize=None)
def _make_sum(nparts, n, c, r):
    def body(*refs):
        x = refs[0][...]
        for p in refs[1:nparts]:
            x = x + p[...]
        refs[nparts][...] = x

    in_specs = [pl.BlockSpec((r, c), lambda i: (i, 0)) for _ in range(nparts)]
    return pl.pallas_call(body, grid=(n // r,), in_specs=in_specs,
                          out_specs=pl.BlockSpec((r, c), lambda i: (i, 0)),
                          out_shape=jax.ShapeDtypeStruct((n, c), jnp.float32))


@functools.lru_cache(maxsize=None)
def _make_down(n, r):
    def body(p0_r, p1_r, ic_r, w_r, bd_r, a_r, g_r, b_r, o_r):
        xm = (p0_r[...] + p1_r[...]) * ic_r[...]
        z = jnp.dot(xm, w_r[...], preferred_element_type=jnp.float32) + bd_r[...]
        o_r[...] = _gn_gelu(z, a_r[...], g_r[...], b_r[...])

    in_specs = [
        pl.BlockSpec((r, _C0), lambda i: (i, 0)),
        pl.BlockSpec((r, _C0), lambda i: (i, 0)),
        pl.BlockSpec((r, _C0), lambda i: (i, 0)),
        pl.BlockSpec((_C0, _C1), lambda i: (0, 0)),
        pl.BlockSpec((1, _C1), lambda i: (0, 0)),
        pl.BlockSpec((_C1, _C1), lambda i: (0, 0)),
        pl.BlockSpec((1, _C1), lambda i: (0, 0)),
        pl.BlockSpec((1, _C1), lambda i: (0, 0)),
    ]
    return pl.pallas_call(body, grid=(n // r,), in_specs=in_specs,
                          out_specs=pl.BlockSpec((r, _C1), lambda i: (i, 0)),
                          out_shape=jax.ShapeDtypeStruct((n, _C1), jnp.float32))


@functools.lru_cache(maxsize=None)
def _make_up(n, r):
    def body(g_r, w_r, bu_r, a_r, gg_r, gb_r, s_r, o_r):
        z = jnp.dot(g_r[...], w_r[...], preferred_element_type=jnp.float32) + bu_r[...]
        o_r[...] = _gn_gelu(z, a_r[...], gg_r[...], gb_r[...]) + s_r[...]

    in_specs = [
        pl.BlockSpec((r, _C1), lambda i: (i, 0)),
        pl.BlockSpec((_C1, _C0), lambda i: (0, 0)),
        pl.BlockSpec((1, _C0), lambda i: (0, 0)),
        pl.BlockSpec((_C0, _C0), lambda i: (0, 0)),
        pl.BlockSpec((1, _C0), lambda i: (0, 0)),
        pl.BlockSpec((1, _C0), lambda i: (0, 0)),
        pl.BlockSpec((r, _C0), lambda i: (i, 0)),
    ]
    return pl.pallas_call(body, grid=(n // r,), in_specs=in_specs,
                          out_specs=pl.BlockSpec((r, _C0), lambda i: (i, 0)),
                          out_shape=jax.ShapeDtypeStruct((n, _C0), jnp.float32))


@functools.lru_cache(maxsize=None)
def _make_agg(npad, epad, nbuf, kk=_K, c=128):
    rps = npad // 16
    epw = epad // _NW
    nch = epw // kk
    m = 1024 // kk
    nit = nch // m
    mesh = plsc.VectorSubcoreMesh(core_axis_name="c", subcore_axis_name="s")

    scr = [pltpu.VMEM((m, kk), jnp.int32), pltpu.VMEM((m, kk), jnp.int32),
           pltpu.VMEM((m, kk), jnp.int32), pltpu.VMEM((m, kk), jnp.int32)]
    scr += [pltpu.VMEM((kk, c), jnp.float32)] * nbuf
    scr += [pltpu.VMEM_SHARED((npad, c), jnp.float32)]
    scr += [pltpu.SemaphoreType.DMA] * nbuf
    scr += [pltpu.SemaphoreType.DMA]

    @functools.partial(
        pl.kernel, mesh=mesh,
        out_type=jax.ShapeDtypeStruct((2, npad, c), jnp.float32),
        scratch_types=scr,
    )
    def agg(y_hbm, is_hbm, id_hbm, out_hbm, *rest):
        ixs = (rest[0], rest[1])
        ixd = (rest[2], rest[3])
        rows = rest[4:4 + nbuf]
        acc = rest[4 + nbuf]
        sems = rest[5 + nbuf:5 + 2 * nbuf]
        isem = rest[5 + 2 * nbuf]
        cid = lax.axis_index("c")
        sid = lax.axis_index("s")
        w = cid * 16 + sid

        def zfill(i, _):
            rr = i // (c // 16)
            col = (i % (c // 16)) * 16
            for u in range(nbuf):
                rows[u][rr, pl.ds(col, 16)] = jnp.zeros((16,), jnp.float32)
            return 0
        lax.fori_loop(0, 32 * (c // 16), zfill, 0)

        nz = rps // 32

        def zacc(j, _):
            zs = [pltpu.async_copy(
                rows[u].at[pl.ds(0, 32), :],
                acc.at[pl.ds(sid * rps + (j * nbuf + u) * 32, 32), :],
                sems[u]) for u in range(nbuf)]
            for z in zs:
                z.wait()
            return 0
        lax.fori_loop(0, nz // nbuf, zacc, 0)
        for t in range((nz // nbuf) * nbuf, nz):
            pltpu.sync_copy(rows[0].at[pl.ds(0, 32), :],
                            acc.at[pl.ds(sid * rps + t * 32, 32), :])
        plsc.subcore_barrier()

        def load_idx(i, slot):
            ro = w * nch + i * m
            a = pltpu.async_copy(is_hbm.at[pl.ds(ro, m), :], ixs[slot], isem)
            b = pltpu.async_copy(id_hbm.at[pl.ds(ro, m), :], ixd[slot], isem)
            return a, b

        a0, b0 = load_idx(0, 0)
        a0.wait()
        b0.wait()

        def body(i, _):
            @pl.when(i + 1 < nit)
            def _():
                ro = w * nch + (i + 1) * m
                pltpu.async_copy(is_hbm.at[pl.ds(ro, m), :], ixs[1], isem)
                pltpu.async_copy(id_hbm.at[pl.ds(ro, m), :], ixd[1], isem)

            gets = {}
            puts = {}
            for u in range(nbuf):
                gets[u] = pltpu.async_copy(y_hbm.at[ixs[0].at[u]], rows[u],
                                           sems[u])
            for k in range(m):
                u = k % nbuf
                gets[k].wait()
                puts[k] = pltpu.async_copy(rows[u], acc.at[ixd[0].at[k]],
                                           sems[u], add=True)
                if k + nbuf < m:
                    puts[k].wait()
                    gets[k + nbuf] = pltpu.async_copy(
                        y_hbm.at[ixs[0].at[k + nbuf]], rows[u], sems[u])
            for k in range(max(0, m - nbuf), m):
                puts[k].wait()

            @pl.when(i + 1 < nit)
            def _():
                pltpu.make_async_copy(is_hbm.at[pl.ds(0, m), :], ixs[1], isem).wait()
                pltpu.make_async_copy(id_hbm.at[pl.ds(0, m), :], ixd[1], isem).wait()

                def rot(i2, _):
                    rr = i2 // (kk // 16)
                    col = (i2 % (kk // 16)) * 16
                    ixs[0][rr, pl.ds(col, 16)] = ixs[1][rr, pl.ds(col, 16)]
                    ixd[0][rr, pl.ds(col, 16)] = ixd[1][rr, pl.ds(col, 16)]
                    return 0
                lax.fori_loop(0, 64, rot, 0)
            return 0
        lax.fori_loop(0, nit, body, 0)
        plsc.subcore_barrier()

        pltpu.sync_copy(acc.at[pl.ds(sid * rps, rps), :],
                        out_hbm.at[cid, pl.ds(sid * rps, rps), :])

    return agg


@functools.lru_cache(maxsize=None)
def _make_gather(npad, c=128):
    rpw = npad // _NW
    nch = rpw // _K
    nit = nch // 8
    mesh = plsc.VectorSubcoreMesh(core_axis_name="c", subcore_axis_name="s")

    @functools.partial(
        pl.kernel, mesh=mesh,
        out_type=jax.ShapeDtypeStruct((npad, c), jnp.float32),
        scratch_types=[
            pltpu.VMEM((8, _K), jnp.int32),
            pltpu.VMEM((_K, c), jnp.float32),
            pltpu.VMEM((_K, c), jnp.float32),
            pltpu.VMEM((_K, c), jnp.float32),
            pltpu.VMEM((_K, c), jnp.float32),
            pltpu.SemaphoreType.DMA,
            pltpu.SemaphoreType.DMA,
            pltpu.SemaphoreType.DMA,
            pltpu.SemaphoreType.DMA,
        ],
    )
    def gat(tab_hbm, idx_hbm, out_hbm, ixs, r0, r1, r2, r3, s0, s1, s2, s3):
        cid = lax.axis_index("c")
        sid = lax.axis_index("s")
        w = cid * 16 + sid
        rows = (r0, r1, r2, r3)
        sems = (s0, s1, s2, s3)

        def body(i, _):
            ro = w * nch + i * 8
            pltpu.sync_copy(idx_hbm.at[pl.ds(ro, 8), :], ixs)
            for g in range(2):
                gets = [pltpu.async_copy(tab_hbm.at[ixs.at[g * 4 + u]],
                                         rows[u], sems[u])
                        for u in range(4)]
                puts = []
                for u in range(4):
                    gets[u].wait()
                    puts.append(pltpu.async_copy(
                        rows[u],
                        out_hbm.at[pl.ds((ro + g * 4 + u) * _K, _K), :],
                        sems[u]))
                for u in range(4):
                    puts[u].wait()
            return 0
        lax.fori_loop(0, nit, body, 0)

    return gat


def _wcat(p):
    cin = p["Ws"].shape[0]
    cout = p["Ws"].shape[1]
    w = jnp.concatenate([p["Ws"], p["W"].transpose(1, 0, 2).reshape(cin, _T * cout)], axis=1)
    b = jnp.concatenate([p["b"], jnp.zeros((_T * cout,), jnp.float32)]).reshape(1, (_T + 1) * cout)
    return w, b


def _pad_i32(x, target, fill_lo, fill_n):
    npd = target - x.shape[0]
    pad = fill_lo + jnp.arange(npd, dtype=jnp.int32) % fill_n
    return jnp.concatenate([x, pad])


def _resblock(parts, p, n, c, r, npad, epad, nbuf, kk, s_idx, d_idx, a_np):
    a = jnp.asarray(a_np)
    agg = _make_agg(npad, epad, nbuf, kk)

    w1, b1 = _wcat(p["c1"])
    g1 = p["n1"]["g"].reshape(1, c)
    gb1 = p["n1"]["b"].reshape(1, c)
    if len(parts) == 1:
        xin = parts[0]
        base1, y1 = _make_prep(1, n, c, r, False)(xin, a, g1, gb1, w1, b1)
    else:
        xin, base1, y1 = _make_prep(len(parts), n, c, r, True)(*parts, a, g1, gb1, w1, b1)
    q = agg(y1.reshape(-1, 128), s_idx, d_idx)

    w2, b2 = _wcat(p["c2"])
    g2 = p["n2"]["g"].reshape(1, c)
    gb2 = p["n2"]["b"].reshape(1, c)
    base2, y2 = _make_prep(3, n, c, r, False)(
        q[0].reshape(-1, c), q[1].reshape(-1, c), base1, a, g2, gb2, w2, b2)
    q2 = agg(y2.reshape(-1, 128), s_idx, d_idx)
    return [xin, q2[0].reshape(-1, c), q2[1].reshape(-1, c), base2]


def kernel(data, edge_index_d6, edge_type_d6, edge_index_d5, edge_type_d5,
           child2parent, depth, params):
    del depth
    s6 = _pad_i32(edge_index_d6[0] * _T + edge_type_d6, _EPAD6, 0, 512).reshape(-1, 64)
    d6 = _pad_i32(edge_index_d6[1], _EPAD6, _N6, _NPAD6 - _N6).reshape(-1, 64)
    s5a = edge_index_d5[0] * (2 * _T) + 2 * edge_type_d5
    s5 = _pad_i32(jnp.stack([s5a, s5a + 1], -1).reshape(-1), _EPAD5D, 0, 512).reshape(-1, 128)
    d5a = 2 * edge_index_d5[1]
    d5 = _pad_i32(jnp.stack([d5a, d5a + 1], -1).reshape(-1), _EPAD5D,
                  2 * _N5, _NPAD5D - 2 * _N5).reshape(-1, 128)
    sp = _pad_i32(jnp.arange(_N6, dtype=jnp.int32), _EPADP, 0, 512).reshape(-1, 128)
    dp = _pad_i32(child2parent, _EPADP, _N5, _NPAD5 - _N5).reshape(-1, 128)
    upa = 2 * child2parent
    up_idx = _pad_i32(jnp.stack([upa, upa + 1], -1).reshape(-1), _NPADU, 0, 512).reshape(-1, 128)
    cnt = jax.ops.segment_sum(jnp.ones((_N6,), jnp.float32), child2parent,
                              num_segments=_NPAD5)
    invc = jnp.broadcast_to((1.0 / jnp.clip(cnt, 1.0))[:, None], (_NPAD5, _C0))

    n5e = _NPAD5
    r6, r5 = 1000, 640

    parts = [data]
    for rb in params["enc0"]:
        parts = _resblock(parts, rb, _N6, _C0, r6, _NPAD6, _EPAD6, 4, 64, s6, d6, _A0_np)
    x6 = _make_sum(4, _N6, _C0, r6)(*parts)

    qp = _make_agg(_NPAD5, _EPADP, 4, 128)(x6, sp, dp)
    a1 = jnp.asarray(_A1_np)
    x5 = _make_down(n5e, r5)(
        qp[0], qp[1], invc, params["down"]["W"],
        params["down"]["b"].reshape(1, _C1), a1,
        params["down"]["n"]["g"].reshape(1, _C1),
        params["down"]["n"]["b"].reshape(1, _C1))

    parts = [x5]
    for rb in params["enc1"]:
        parts = _resblock(parts, rb, n5e, _C1, r5, _NPAD5D, _EPAD5D, 4, 128, s5, d5, _A1_np)
    for rb in params["dec0"]:
        parts = _resblock(parts, rb, n5e, _C1, r5, _NPAD5D, _EPAD5D, 4, 128, s5, d5, _A1_np)
    out5 = _make_sum(4, n5e, _C1, r5)(*parts)

    g = _make_gather(_NPADU)(out5.reshape(-1, 128), up_idx).reshape(-1, _C1)
    a0 = jnp.asarray(_A0_np)
    xu = _make_up(_N6, r6)(
        g, params["up"]["W"], params["up"]["b"].reshape(1, _C0), a0,
        params["up"]["n"]["g"].reshape(1, _C0),
        params["up"]["n"]["b"].reshape(1, _C0), x6)

    parts = [xu]
    for rb in params["dec1"]:
        parts = _resblock(parts, rb, _N6, _C0, r6, _NPAD6, _EPAD6, 4, 64, s6, d6, _A0_np)
    return _make_sum(4, _N6, _C0, r6)(*parts)


# speedup vs baseline: 29.5154x; 1.0112x over previous
import functools

import numpy as np
import jax
import jax.numpy as jnp
from jax import lax
from jax.experimental import pallas as pl
from jax.experimental.pallas import tpu as pltpu
from jax.experimental.pallas import tpu_sc as plsc

_N6 = 10000
_N5 = 2500
_E6 = 320000
_E5 = 80000
_C0 = 128
_C1 = 256
_G = 32
_T = 7

_NW = 32
_K = 128

_NPAD6 = 10240
_NPAD5 = 2560
_NPAD5D = 5120
_EPAD6 = 327680
_EPAD5D = 163840
_EPADP = 32768
_NPADU = 32768

_ZR = 32


def _gn_mats():
    a0 = np.kron(np.eye(_G, dtype=np.float32), np.full((4, 4), 0.25, np.float32))
    a1 = np.kron(np.eye(_G, dtype=np.float32), np.full((8, 8), 0.125, np.float32))
    return a0, a1


_A0_np, _A1_np = _gn_mats()


def _gn_gelu(x, a, g, b):
    mu = jnp.dot(x, a, preferred_element_type=jnp.float32)
    ex2 = jnp.dot(x * x, a, preferred_element_type=jnp.float32)
    var = ex2 - mu * mu
    xn = (x - mu) * lax.rsqrt(var + 1e-5)
    return jax.nn.gelu(xn * g + b)


@functools.lru_cache(maxsize=None)
def _make_prep(nparts, n, c, r, emit_xin):
    wc = (_T + 1) * c

    def body(*refs):
        parts = refs[:nparts]
        a_r, g_r, b_r, w_r, bc_r = refs[nparts:nparts + 5]
        outs = refs[nparts + 5:]
        x = parts[0][...]
        for p in parts[1:]:
            x = x + p[...]
        u = _gn_gelu(x, a_r[...], g_r[...], b_r[...])
        z = jnp.dot(u, w_r[...], preferred_element_type=jnp.float32) + bc_r[...]
        if emit_xin:
            outs[0][...] = x
            outs = outs[1:]
        outs[0][...] = z[:, :c]
        outs[1][...] = z[:, c:]

    in_specs = [pl.BlockSpec((r, c), lambda i: (i, 0)) for _ in range(nparts)]
    in_specs += [
        pl.BlockSpec((c, c), lambda i: (0, 0)),
        pl.BlockSpec((1, c), lambda i: (0, 0)),
        pl.BlockSpec((1, c), lambda i: (0, 0)),
        pl.BlockSpec((c, wc), lambda i: (0, 0)),
        pl.BlockSpec((1, wc), lambda i: (0, 0)),
    ]
    out_shape = []
    out_specs = []
    if emit_xin:
        out_shape.append(jax.ShapeDtypeStruct((n, c), jnp.float32))
        out_specs.append(pl.BlockSpec((r, c), lambda i: (i, 0)))
    out_shape += [jax.ShapeDtypeStruct((n, c), jnp.float32),
                  jax.ShapeDtypeStruct((n, _T * c), jnp.float32)]
    out_specs += [pl.BlockSpec((r, c), lambda i: (i, 0)),
                  pl.BlockSpec((r, _T * c), lambda i: (i, 0))]
    return pl.pallas_call(body, grid=(n // r,), in_specs=in_specs,
                          out_specs=out_specs, out_shape=out_shape)


@functools.lru_cache(maxsize=None)
def _make_sum(nparts, n, c, r):
    def body(*refs):
        x = refs[0][...]
        for p in refs[1:nparts]:
            x = x + p[...]
        refs[nparts][...] = x

    in_specs = [pl.BlockSpec((r, c), lambda i: (i, 0)) for _ in range(nparts)]
    return pl.pallas_call(body, grid=(n // r,), in_specs=in_specs,
                          out_specs=pl.BlockSpec((r, c), lambda i: (i, 0)),
                          out_shape=jax.ShapeDtypeStruct((n, c), jnp.float32))


@functools.lru_cache(maxsize=None)
def _make_down(n, r):
    def body(p0_r, p1_r, ic_r, w_r, bd_r, a_r, g_r, b_r, o_r):
        xm = (p0_r[...] + p1_r[...]) * ic_r[...]
        z = jnp.dot(xm, w_r[...], preferred_element_type=jnp.float32) + bd_r[...]
        o_r[...] = _gn_gelu(z, a_r[...], g_r[...], b_r[...])

    in_specs = [
        pl.BlockSpec((r, _C0), lambda i: (i, 0)),
        pl.BlockSpec((r, _C0), lambda i: (i, 0)),
        pl.BlockSpec((r, _C0), lambda i: (i, 0)),
        pl.BlockSpec((_C0, _C1), lambda i: (0, 0)),
        pl.BlockSpec((1, _C1), lambda i: (0, 0)),
        pl.BlockSpec((_C1, _C1), lambda i: (0, 0)),
        pl.BlockSpec((1, _C1), lambda i: (0, 0)),
        pl.BlockSpec((1, _C1), lambda i: (0, 0)),
    ]
    return pl.pallas_call(body, grid=(n // r,), in_specs=in_specs,
                          out_specs=pl.BlockSpec((r, _C1), lambda i: (i, 0)),
                          out_shape=jax.ShapeDtypeStruct((n, _C1), jnp.float32))


@functools.lru_cache(maxsize=None)
def _make_up(n, r):
    def body(g_r, w_r, bu_r, a_r, gg_r, gb_r, s_r, o_r):
        z = jnp.dot(g_r[...], w_r[...], preferred_element_type=jnp.float32) + bu_r[...]
        o_r[...] = _gn_gelu(z, a_r[...], gg_r[...], gb_r[...]) + s_r[...]

    in_specs = [
        pl.BlockSpec((r, _C1), lambda i: (i, 0)),
        pl.BlockSpec((_C1, _C0), lambda i: (0, 0)),
        pl.BlockSpec((1, _C0), lambda i: (0, 0)),
        pl.BlockSpec((_C0, _C0), lambda i: (0, 0)),
        pl.BlockSpec((1, _C0), lambda i: (0, 0)),
        pl.BlockSpec((1, _C0), lambda i: (0, 0)),
        pl.BlockSpec((r, _C0), lambda i: (i, 0)),
    ]
    return pl.pallas_call(body, grid=(n // r,), in_specs=in_specs,
                          out_specs=pl.BlockSpec((r, _C0), lambda i: (i, 0)),
                          out_shape=jax.ShapeDtypeStruct((n, _C0), jnp.float32))


@functools.lru_cache(maxsize=None)
def _make_agg(npad, epad, nbuf, kk=_K, c=128):
    rps = npad // 16
    epw = epad // _NW
    nch = epw // kk
    m = 1024 // kk
    nit = nch // m
    mesh = plsc.VectorSubcoreMesh(core_axis_name="c", subcore_axis_name="s")

    scr = [pltpu.VMEM((m, kk), jnp.int32), pltpu.VMEM((m, kk), jnp.int32),
           pltpu.VMEM((m, kk), jnp.int32), pltpu.VMEM((m, kk), jnp.int32)]
    scr += [pltpu.VMEM((kk, c), jnp.float32)] * nbuf
    scr += [pltpu.VMEM_SHARED((npad, c), jnp.float32)]
    scr += [pltpu.SemaphoreType.DMA] * nbuf
    scr += [pltpu.SemaphoreType.DMA]

    @functools.partial(
        pl.kernel, mesh=mesh,
        out_type=jax.ShapeDtypeStruct((2, npad, c), jnp.float32),
        scratch_types=scr,
    )
    def agg(y_hbm, is_hbm, id_hbm, out_hbm, *rest):
        ixs = (rest[0], rest[1])
        ixd = (rest[2], rest[3])
        rows = rest[4:4 + nbuf]
        acc = rest[4 + nbuf]
        sems = rest[5 + nbuf:5 + 2 * nbuf]
        isem = rest[5 + 2 * nbuf]
        cid = lax.axis_index("c")
        sid = lax.axis_index("s")
        w = cid * 16 + sid

        def zfill(i, _):
            rr = i // (c // 16)
            col = (i % (c // 16)) * 16
            for u in range(nbuf):
                rows[u][rr, pl.ds(col, 16)] = jnp.zeros((16,), jnp.float32)
            return 0
        lax.fori_loop(0, 32 * (c // 16), zfill, 0)

        nz = rps // 32

        def zacc(j, _):
            zs = [pltpu.async_copy(
                rows[u].at[pl.ds(0, 32), :],
                acc.at[pl.ds(sid * rps + (j * nbuf + u) * 32, 32), :],
                sems[u]) for u in range(nbuf)]
            for z in zs:
                z.wait()
            return 0
        lax.fori_loop(0, nz // nbuf, zacc, 0)
        for t in range((nz // nbuf) * nbuf, nz):
            pltpu.sync_copy(rows[0].at[pl.ds(0, 32), :],
                            acc.at[pl.ds(sid * rps + t * 32, 32), :])
        plsc.subcore_barrier()

        def load_idx(i, slot):
            ro = w * nch + i * m
            a = pltpu.async_copy(is_hbm.at[pl.ds(ro, m), :], ixs[slot], isem)
            b = pltpu.async_copy(id_hbm.at[pl.ds(ro, m), :], ixd[slot], isem)
            return a, b

        a0, b0 = load_idx(0, 0)
        a0.wait()
        b0.wait()

        def body(i, _):
            @pl.when(i + 1 < nit)
            def _():
                ro = w * nch + (i + 1) * m
                pltpu.async_copy(is_hbm.at[pl.ds(ro, m), :], ixs[1], isem)
                pltpu.async_copy(id_hbm.at[pl.ds(ro, m), :], ixd[1], isem)

            gets = {}
            puts = {}
            for u in range(nbuf):
                gets[u] = pltpu.async_copy(y_hbm.at[ixs[0].at[u]], rows[u],
                                           sems[u])
            for k in range(m):
                u = k % nbuf
                gets[k].wait()
                puts[k] = pltpu.async_copy(rows[u], acc.at[ixd[0].at[k]],
                                           sems[u], add=True)
                if k + nbuf < m:
                    puts[k].wait()
                    gets[k + nbuf] = pltpu.async_copy(
                        y_hbm.at[ixs[0].at[k + nbuf]], rows[u], sems[u])
            for k in range(max(0, m - nbuf), m):
                puts[k].wait()

            @pl.when(i + 1 < nit)
            def _():
                pltpu.make_async_copy(is_hbm.at[pl.ds(0, m), :], ixs[1], isem).wait()
                pltpu.make_async_copy(id_hbm.at[pl.ds(0, m), :], ixd[1], isem).wait()

                def rot(i2, _):
                    rr = i2 // (kk // 16)
                    col = (i2 % (kk // 16)) * 16
                    ixs[0][rr, pl.ds(col, 16)] = ixs[1][rr, pl.ds(col, 16)]
                    ixd[0][rr, pl.ds(col, 16)] = ixd[1][rr, pl.ds(col, 16)]
                    return 0
                lax.fori_loop(0, 64, rot, 0)
            return 0
        lax.fori_loop(0, nit, body, 0)
        plsc.subcore_barrier()

        pltpu.sync_copy(acc.at[pl.ds(sid * rps, rps), :],
                        out_hbm.at[cid, pl.ds(sid * rps, rps), :])

    return agg


@functools.lru_cache(maxsize=None)
def _make_gather(npad, c=128):
    rpw = npad // _NW
    nch = rpw // _K
    nit = nch // 8
    mesh = plsc.VectorSubcoreMesh(core_axis_name="c", subcore_axis_name="s")

    @functools.partial(
        pl.kernel, mesh=mesh,
        out_type=jax.ShapeDtypeStruct((npad, c), jnp.float32),
        scratch_types=[
            pltpu.VMEM((8, _K), jnp.int32),
            pltpu.VMEM((_K, c), jnp.float32),
            pltpu.VMEM((_K, c), jnp.float32),
            pltpu.VMEM((_K, c), jnp.float32),
            pltpu.VMEM((_K, c), jnp.float32),
            pltpu.SemaphoreType.DMA,
            pltpu.SemaphoreType.DMA,
            pltpu.SemaphoreType.DMA,
            pltpu.SemaphoreType.DMA,
        ],
    )
    def gat(tab_hbm, idx_hbm, out_hbm, ixs, r0, r1, r2, r3, s0, s1, s2, s3):
        cid = lax.axis_index("c")
        sid = lax.axis_index("s")
        w = cid * 16 + sid
        rows = (r0, r1, r2, r3)
        sems = (s0, s1, s2, s3)

        def body(i, _):
            ro = w * nch + i * 8
            pltpu.sync_copy(idx_hbm.at[pl.ds(ro, 8), :], ixs)
            for g in range(2):
                gets = [pltpu.async_copy(tab_hbm.at[ixs.at[g * 4 + u]],
                                         rows[u], sems[u])
                        for u in range(4)]
                puts = []
                for u in range(4):
                    gets[u].wait()
                    puts.append(pltpu.async_copy(
                        rows[u],
                        out_hbm.at[pl.ds((ro + g * 4 + u) * _K, _K), :],
                        sems[u]))
                for u in range(4):
                    puts[u].wait()
            return 0
        lax.fori_loop(0, nit, body, 0)

    return gat


def _wcat(p):
    cin = p["Ws"].shape[0]
    cout = p["Ws"].shape[1]
    w = jnp.concatenate([p["Ws"], p["W"].transpose(1, 0, 2).reshape(cin, _T * cout)], axis=1)
    b = jnp.concatenate([p["b"], jnp.zeros((_T * cout,), jnp.float32)]).reshape(1, (_T + 1) * cout)
    return w, b


def _pad_i32(x, target, fill_lo, fill_n):
    npd = target - x.shape[0]
    pad = fill_lo + jnp.arange(npd, dtype=jnp.int32) % fill_n
    return jnp.concatenate([x, pad])


def _resblock(parts, p, n, c, r, npad, epad, nbuf, kk, s_idx, d_idx, a_np):
    a = jnp.asarray(a_np)
    agg = _make_agg(npad, epad, nbuf, kk)

    w1, b1 = _wcat(p["c1"])
    g1 = p["n1"]["g"].reshape(1, c)
    gb1 = p["n1"]["b"].reshape(1, c)
    if len(parts) == 1:
        xin = parts[0]
        base1, y1 = _make_prep(1, n, c, r, False)(xin, a, g1, gb1, w1, b1)
    else:
        xin, base1, y1 = _make_prep(len(parts), n, c, r, True)(*parts, a, g1, gb1, w1, b1)
    q = agg(y1.reshape(-1, 128), s_idx, d_idx)

    w2, b2 = _wcat(p["c2"])
    g2 = p["n2"]["g"].reshape(1, c)
    gb2 = p["n2"]["b"].reshape(1, c)
    base2, y2 = _make_prep(3, n, c, r, False)(
        q[0].reshape(-1, c), q[1].reshape(-1, c), base1, a, g2, gb2, w2, b2)
    q2 = agg(y2.reshape(-1, 128), s_idx, d_idx)
    return [xin, q2[0].reshape(-1, c), q2[1].reshape(-1, c), base2]


def kernel(data, edge_index_d6, edge_type_d6, edge_index_d5, edge_type_d5,
           child2parent, depth, params):
    del depth
    s6 = _pad_i32(edge_index_d6[0] * _T + edge_type_d6, _EPAD6, 0, 512).reshape(-1, 64)
    d6 = _pad_i32(edge_index_d6[1], _EPAD6, _N6, _NPAD6 - _N6).reshape(-1, 64)
    s5a = edge_index_d5[0] * (2 * _T) + 2 * edge_type_d5
    s5 = _pad_i32(jnp.stack([s5a, s5a + 1], -1).reshape(-1), _EPAD5D, 0, 512).reshape(-1, 64)
    d5a = 2 * edge_index_d5[1]
    d5 = _pad_i32(jnp.stack([d5a, d5a + 1], -1).reshape(-1), _EPAD5D,
                  2 * _N5, _NPAD5D - 2 * _N5).reshape(-1, 64)
    sp = _pad_i32(jnp.arange(_N6, dtype=jnp.int32), _EPADP, 0, 512).reshape(-1, 128)
    dp = _pad_i32(child2parent, _EPADP, _N5, _NPAD5 - _N5).reshape(-1, 128)
    upa = 2 * child2parent
    up_idx = _pad_i32(jnp.stack([upa, upa + 1], -1).reshape(-1), _NPADU, 0, 512).reshape(-1, 128)
    cnt = jax.ops.segment_sum(jnp.ones((_N6,), jnp.float32), child2parent,
                              num_segments=_NPAD5)
    invc = jnp.broadcast_to((1.0 / jnp.clip(cnt, 1.0))[:, None], (_NPAD5, _C0))

    n5e = _NPAD5
    r6, r5 = 1000, 640

    parts = [data]
    for rb in params["enc0"]:
        parts = _resblock(parts, rb, _N6, _C0, r6, _NPAD6, _EPAD6, 5, 64, s6, d6, _A0_np)
    x6 = _make_sum(4, _N6, _C0, r6)(*parts)

    qp = _make_agg(_NPAD5, _EPADP, 4, 128)(x6, sp, dp)
    a1 = jnp.asarray(_A1_np)
    x5 = _make_down(n5e, r5)(
        qp[0], qp[1], invc, params["down"]["W"],
        params["down"]["b"].reshape(1, _C1), a1,
        params["down"]["n"]["g"].reshape(1, _C1),
        params["down"]["n"]["b"].reshape(1, _C1))

    parts = [x5]
    for rb in params["enc1"]:
        parts = _resblock(parts, rb, n5e, _C1, r5, _NPAD5D, _EPAD5D, 5, 64, s5, d5, _A1_np)
    for rb in params["dec0"]:
        parts = _resblock(parts, rb, n5e, _C1, r5, _NPAD5D, _EPAD5D, 5, 64, s5, d5, _A1_np)
    out5 = _make_sum(4, n5e, _C1, r5)(*parts)

    g = _make_gather(_NPADU)(out5.reshape(-1, 128), up_idx).reshape(-1, _C1)
    a0 = jnp.asarray(_A0_np)
    xu = _make_up(_N6, r6)(
        g, params["up"]["W"], params["up"]["b"].reshape(1, _C0), a0,
        params["up"]["n"]["g"].reshape(1, _C0),
        params["up"]["n"]["b"].reshape(1, _C0), x6)

    parts = [xu]
    for rb in params["dec1"]:
        parts = _resblock(parts, rb, _N6, _C0, r6, _NPAD6, _EPAD6, 5, 64, s6, d6, _A0_np)
    return _make_sum(4, _N6, _C0, r6)(*parts)
